# baseline (speedup 1.0000x reference)
import functools

import jax
import jax.numpy as jnp
from jax import lax
from jax.experimental import pallas as pl
from jax.experimental.pallas import tpu as pltpu
from jax.experimental.pallas import tpu_sc as plsc

_NC = 2
_NS = 16
_NW = _NC * _NS
_L = 16
_B = 80


def _prep_body(x_ref, w_ref, asrc_ref, adst_ref, xp_ref, as_ref, ad_ref, m_ref):
    xp = jnp.dot(x_ref[...], w_ref[...], preferred_element_type=jnp.float32)
    xp_ref[...] = xp
    a_s = jnp.sum(xp * asrc_ref[...][None, :], axis=1)
    a_d = jnp.sum(xp * adst_ref[...][None, :], axis=1)
    as_ref[...] = a_s
    ad_ref[...] = a_d
    z = jnp.max(a_s) + jnp.max(a_d)
    m = jnp.where(z >= 0.0, z, 0.2 * z)
    m_ref[...] = jnp.full((16,), m, jnp.float32)


def _prep(x, W, att_src, att_dst):
    n, d = x.shape
    c = W.shape[1]
    return pl.pallas_call(
        _prep_body,
        out_shape=(
            jax.ShapeDtypeStruct((n, c), jnp.float32),
            jax.ShapeDtypeStruct((n,), jnp.float32),
            jax.ShapeDtypeStruct((n,), jnp.float32),
            jax.ShapeDtypeStruct((16,), jnp.float32),
        ),
    )(x, W, att_src, att_dst)


def _edge_body(n, e, c, xp_hbm, as_hbm, ad_hbm, m_hbm, packed_hbm,
               num_out, den_out,
               idx2, rows_v0, rows_v1, sb0, db0, exb0, dstb0,
               sb1, db1, exb1, dstb1, m_v,
               num_sh, den_sh, gsem0, gsem1, lsem0, lsem1,
               ssem0, ssem1, dsem0, dsem1):
    ci = lax.axis_index("c")
    si = lax.axis_index("s")
    wid = si * _NC + ci
    e_per_tile = e // _NW
    n_chunks = e_per_tile // _B
    rpt = (n // _NS) // 8 * 8
    rem_rows = n - rpt * _NS

    pltpu.sync_copy(m_hbm, m_v)
    m = m_v[...]

    zeros16 = jnp.zeros((_L,), jnp.float32)
    for i in range(_B // _L):
        exb0[pl.ds(i * _L, _L)] = zeros16

    def _zero_rows(i, carry):
        for k in range(c // _L):
            rows_v0[i, pl.ds(k * _L, _L)] = zeros16
        return carry
    lax.fori_loop(0, _B, _zero_rows, 0)

    row0 = si * rpt
    full, rem = rpt // _B, rpt % _B
    for t in range(full):
        pltpu.sync_copy(rows_v0, num_sh.at[pl.ds(row0 + t * _B, _B)])
        pltpu.sync_copy(exb0, den_sh.at[pl.ds(row0 + t * _B, _B)])
    if rem:
        pltpu.sync_copy(rows_v0.at[pl.ds(0, rem)],
                        num_sh.at[pl.ds(row0 + full * _B, rem)])
        pltpu.sync_copy(exb0.at[pl.ds(0, rem)],
                        den_sh.at[pl.ds(row0 + full * _B, rem)])
    if rem_rows:
        @pl.when(si == _NS - 1)
        def _zero_tail():
            pltpu.sync_copy(rows_v0.at[pl.ds(0, rem_rows)],
                            num_sh.at[pl.ds(n - rem_rows, rem_rows)])
            pltpu.sync_copy(exb0.at[pl.ds(0, rem_rows)],
                            den_sh.at[pl.ds(n - rem_rows, rem_rows)])
    pltpu.sync_copy(packed_hbm.at[pl.ds(wid * n_chunks * 2 * _B,
                                        n_chunks * 2 * _B)], idx2)
    plsc.subcore_barrier()

    bufs = ((sb0, db0, exb0, dstb0, rows_v0, gsem0, lsem0, ssem0, dsem0),
            (sb1, db1, exb1, dstb1, rows_v1, gsem1, lsem1, ssem1, dsem1))

    def _issue(kn, nxt, wait_scat):
        sb, db, exb, dstb, rowsb, gsem, lsem, ssem, dsem = bufs[nxt]
        if wait_scat:
            pltpu.make_async_copy(rowsb, num_sh.at[dstb], ssem).wait()
            pltpu.make_async_copy(exb, den_sh.at[dstb], dsem).wait()
        sidx = idx2.at[pl.ds(kn * 2 * _B, _B)]
        didx = idx2.at[pl.ds(kn * 2 * _B + _B, _B)]
        pltpu.async_copy(xp_hbm.at[sidx], rowsb, gsem)
        pltpu.async_copy(as_hbm.at[sidx], sb, lsem)
        pltpu.async_copy(ad_hbm.at[didx], db, lsem)

    def _compute(k, cur):
        sb, db, exb, dstb, rowsb, gsem, lsem, ssem, dsem = bufs[cur]
        sidx = idx2.at[pl.ds(k * 2 * _B, _B)]
        didx = idx2.at[pl.ds(k * 2 * _B + _B, _B)]
        pltpu.make_async_copy(as_hbm.at[sidx], sb, lsem).wait()
        pltpu.make_async_copy(ad_hbm.at[didx], db, lsem).wait()
        for j in range(_B // _L):
            z = sb[pl.ds(j * _L, _L)] + db[pl.ds(j * _L, _L)]
            e16 = jnp.where(z >= 0.0, z, jnp.float32(0.2) * z)
            exb[pl.ds(j * _L, _L)] = jnp.exp(e16 - m)
            dstb[pl.ds(j * _L, _L)] = idx2[pl.ds(k * 2 * _B + _B + j * _L, _L)]
        pltpu.async_copy(exb, den_sh.at[dstb], dsem, add=True)
        pltpu.make_async_copy(xp_hbm.at[sidx], rowsb, gsem).wait()
        for g in range(_B // _L):
            ex16 = exb[pl.ds(g * _L, _L)]
            for jj in range(_L):
                row = g * _L + jj
                exj = jnp.full((_L,), ex16[jj], jnp.float32)
                for kk in range(c // _L):
                    rowsb[row, pl.ds(kk * _L, _L)] = (
                        rowsb[row, pl.ds(kk * _L, _L)] * exj)
        pltpu.async_copy(rowsb, num_sh.at[dstb], ssem, add=True)

    _issue(0, 0, False)
    _issue(1, 1, False)
    _compute(0, 0)

    def _pair(i, carry):
        k0 = 1 + 2 * i
        _issue(k0 + 1, 0, True)
        _compute(k0, 1)
        _issue(k0 + 2, 1, True)
        _compute(k0 + 1, 0)
        return carry
    lax.fori_loop(0, (n_chunks - 3) // 2, _pair, 0)
    _issue(n_chunks - 1, 0, True)
    _compute(n_chunks - 2, 1)
    _compute(n_chunks - 1, 0)
    pltpu.make_async_copy(rows_v0, num_sh.at[dstb0], ssem0).wait()
    pltpu.make_async_copy(rows_v1, num_sh.at[dstb1], ssem1).wait()
    pltpu.make_async_copy(exb0, den_sh.at[dstb0], dsem0).wait()
    pltpu.make_async_copy(exb1, den_sh.at[dstb1], dsem1).wait()
    plsc.subcore_barrier()

    pltpu.sync_copy(num_sh.at[pl.ds(row0, rpt)],
                    num_out.at[ci, pl.ds(row0, rpt)])
    if rem_rows:
        @pl.when(si == _NS - 1)
        def _wb_tail():
            pltpu.sync_copy(num_sh.at[pl.ds(n - rem_rows, rem_rows)],
                            num_out.at[ci, pl.ds(n - rem_rows, rem_rows)])
    dbase = ci * n
    for t in range(full):
        pltpu.sync_copy(den_sh.at[pl.ds(row0 + t * _B, _B)], exb0)
        pltpu.sync_copy(exb0, den_out.at[pl.ds(dbase + row0 + t * _B, _B)])
    if rem:
        pltpu.sync_copy(den_sh.at[pl.ds(row0 + full * _B, rem)],
                        exb0.at[pl.ds(0, rem)])
        pltpu.sync_copy(exb0.at[pl.ds(0, rem)],
                        den_out.at[pl.ds(dbase + row0 + full * _B, rem)])
    if rem_rows:
        @pl.when(si == _NS - 1)
        def _wb_dtail():
            pltpu.sync_copy(den_sh.at[pl.ds(n - rem_rows, rem_rows)],
                            exb0.at[pl.ds(0, rem_rows)])
            pltpu.sync_copy(exb0.at[pl.ds(0, rem_rows)],
                            den_out.at[pl.ds(dbase + n - rem_rows, rem_rows)])


def _edge(xp, a_s, a_d, m16, packed, n_chunks):
    n, c = xp.shape
    e = _NW * n_chunks * _B
    mesh = plsc.VectorSubcoreMesh(core_axis_name="c", subcore_axis_name="s")
    kern = pl.kernel(
        functools.partial(_edge_body, n, e, c),
        out_type=(
            jax.ShapeDtypeStruct((_NC, n, c), jnp.float32),
            jax.ShapeDtypeStruct((_NC * n,), jnp.float32),
        ),
        mesh=mesh,
        compiler_params=pltpu.CompilerParams(needs_layout_passes=False),
        scratch_types=[
            pltpu.VMEM((n_chunks * 2 * _B,), jnp.int32),
            pltpu.VMEM((_B, c), jnp.float32),
            pltpu.VMEM((_B, c), jnp.float32),
            pltpu.VMEM((_B,), jnp.float32),
            pltpu.VMEM((_B,), jnp.float32),
            pltpu.VMEM((_B,), jnp.float32),
            pltpu.VMEM((_B,), jnp.int32),
            pltpu.VMEM((_B,), jnp.float32),
            pltpu.VMEM((_B,), jnp.float32),
            pltpu.VMEM((_B,), jnp.float32),
            pltpu.VMEM((_B,), jnp.int32),
            pltpu.VMEM((16,), jnp.float32),
            pltpu.VMEM_SHARED((n, c), jnp.float32),
            pltpu.VMEM_SHARED((n,), jnp.float32),
            pltpu.SemaphoreType.DMA,
            pltpu.SemaphoreType.DMA,
            pltpu.SemaphoreType.DMA,
            pltpu.SemaphoreType.DMA,
            pltpu.SemaphoreType.DMA,
            pltpu.SemaphoreType.DMA,
            pltpu.SemaphoreType.DMA,
            pltpu.SemaphoreType.DMA,
        ],
    )
    return kern(xp, a_s, a_d, m16, packed)


def _fin_body(num_ref, den_ref, bias_ref, out_ref):
    num = num_ref[0] + num_ref[1]
    den = jnp.sum(den_ref[0], axis=0) + jnp.float32(1e-16)
    o = num / den[:, None] + bias_ref[...][None, :]
    mx = jnp.max(o, axis=1, keepdims=True)
    lse = jnp.log(jnp.sum(jnp.exp(o - mx), axis=1, keepdims=True)) + mx
    out_ref[...] = o - lse


def _finalize(num_part, den_part, bias):
    _, n, c = num_part.shape
    blk = 1000
    grid = (n // blk,)
    den3 = den_part.reshape(_NC, n // blk, blk).transpose(1, 0, 2)
    return pl.pallas_call(
        _fin_body,
        grid=grid,
        in_specs=[
            pl.BlockSpec((_NC, blk, c), lambda i: (0, i, 0)),
            pl.BlockSpec((1, _NC, blk), lambda i: (i, 0, 0)),
            pl.BlockSpec((c,), lambda i: (0,)),
        ],
        out_specs=pl.BlockSpec((blk, c), lambda i: (i, 0)),
        out_shape=jax.ShapeDtypeStruct((n, c), jnp.float32),
    )(num_part, den3, bias)


def kernel(x, W, att_src, att_dst, bias, edge_index):
    e = edge_index.shape[1]
    n_chunks = e // (_NW * _B)
    packed = (edge_index.reshape(2, _NW, n_chunks, _B)
              .transpose(1, 2, 0, 3).reshape(_NW * n_chunks * 2 * _B))
    xp, a_s, a_d, m16 = _prep(x, W, att_src, att_dst)
    num_part, den_part = _edge(xp, a_s, a_d, m16, packed, n_chunks)
    return _finalize(num_part, den_part, bias)


# speedup vs baseline: 1.0503x; 1.0503x over previous
import functools

import jax
import jax.numpy as jnp
from jax import lax
from jax.experimental import pallas as pl
from jax.experimental.pallas import tpu as pltpu
from jax.experimental.pallas import tpu_sc as plsc

_NC = 2
_NS = 16
_NW = _NC * _NS
_L = 16
_B = 80


def _prep_body(x_ref, w_ref, asrc_ref, adst_ref, xp_ref, as_ref, ad_ref, m_ref):
    xp = jnp.dot(x_ref[...], w_ref[...], preferred_element_type=jnp.float32)
    xp_ref[...] = xp
    a_s = jnp.sum(xp * asrc_ref[...][None, :], axis=1)
    a_d = jnp.sum(xp * adst_ref[...][None, :], axis=1)
    as_ref[...] = a_s
    ad_ref[...] = a_d
    z = jnp.max(a_s) + jnp.max(a_d)
    m = jnp.where(z >= 0.0, z, 0.2 * z)
    m_ref[...] = jnp.full((16,), m, jnp.float32)


def _prep(x, W, att_src, att_dst):
    n, d = x.shape
    c = W.shape[1]
    return pl.pallas_call(
        _prep_body,
        out_shape=(
            jax.ShapeDtypeStruct((n, c), jnp.float32),
            jax.ShapeDtypeStruct((n,), jnp.float32),
            jax.ShapeDtypeStruct((n,), jnp.float32),
            jax.ShapeDtypeStruct((16,), jnp.float32),
        ),
    )(x, W, att_src, att_dst)


_HA = 48
_HB = 32


def _edge_body(n, e, c, xp_hbm, as_hbm, ad_hbm, m_hbm, packed_hbm,
               num_out, den_out,
               idx2, rows0, rows1,
               sb0, db0, exb0, dstb0, dha0, dhb0,
               sb1, db1, exb1, dstb1, dha1, dhb1, m_v,
               num_sh, den_sh,
               gsa0, gsb0, ls0, ds0, fsa0, fsb0,
               gsa1, gsb1, ls1, ds1, fsa1, fsb1):
    ci = lax.axis_index("c")
    si = lax.axis_index("s")
    wid = si * _NC + ci
    e_per_tile = e // _NW
    n_chunks = e_per_tile // _B
    rpt = (n // _NS) // 8 * 8
    rem_rows = n - rpt * _NS

    pltpu.sync_copy(m_hbm, m_v)
    m = m_v[...]

    zeros16 = jnp.zeros((_L,), jnp.float32)
    for i in range(_B // _L):
        exb0[pl.ds(i * _L, _L)] = zeros16

    def _zero_rows(i, carry):
        for k in range(c // _L):
            rows0[i, pl.ds(k * _L, _L)] = zeros16
        return carry
    lax.fori_loop(0, _B, _zero_rows, 0)

    row0 = si * rpt
    full, rem = rpt // _B, rpt % _B
    for t in range(full):
        pltpu.sync_copy(rows0, num_sh.at[pl.ds(row0 + t * _B, _B)])
        pltpu.sync_copy(exb0, den_sh.at[pl.ds(row0 + t * _B, _B)])
    if rem:
        pltpu.sync_copy(rows0.at[pl.ds(0, rem)],
                        num_sh.at[pl.ds(row0 + full * _B, rem)])
        pltpu.sync_copy(exb0.at[pl.ds(0, rem)],
                        den_sh.at[pl.ds(row0 + full * _B, rem)])
    if rem_rows:
        @pl.when(si == _NS - 1)
        def _zero_tail():
            pltpu.sync_copy(rows0.at[pl.ds(0, rem_rows)],
                            num_sh.at[pl.ds(n - rem_rows, rem_rows)])
            pltpu.sync_copy(exb0.at[pl.ds(0, rem_rows)],
                            den_sh.at[pl.ds(n - rem_rows, rem_rows)])
    pltpu.sync_copy(packed_hbm.at[pl.ds(wid * n_chunks * 2 * _B,
                                        n_chunks * 2 * _B)], idx2)
    plsc.subcore_barrier()

    bufs = ((sb0, db0, exb0, dstb0, dha0, dhb0, rows0,
             gsa0, gsb0, ls0, ds0, fsa0, fsb0),
            (sb1, db1, exb1, dstb1, dha1, dhb1, rows1,
             gsa1, gsb1, ls1, ds1, fsa1, fsb1))

    def _issue(kn, nxt, wait_prev):
        (sb, db, exb, dstb, dha, dhb, rows,
         gsa, gsb, ls, ds, fsa, fsb) = bufs[nxt]
        if wait_prev:
            pltpu.make_async_copy(rows.at[pl.ds(0, _HA)],
                                  num_sh.at[dha], fsa).wait()
            pltpu.make_async_copy(rows.at[pl.ds(_HA, _HB)],
                                  num_sh.at[dhb], fsb).wait()
            pltpu.make_async_copy(exb, den_sh.at[dstb], ds).wait()
        sidxa = idx2.at[pl.ds(kn * 2 * _B, _HA)]
        sidxb = idx2.at[pl.ds(kn * 2 * _B + _HA, _HB)]
        sidx = idx2.at[pl.ds(kn * 2 * _B, _B)]
        didx = idx2.at[pl.ds(kn * 2 * _B + _B, _B)]
        pltpu.async_copy(xp_hbm.at[sidxa], rows.at[pl.ds(0, _HA)], gsa)
        pltpu.async_copy(xp_hbm.at[sidxb], rows.at[pl.ds(_HA, _HB)], gsb)
        pltpu.async_copy(as_hbm.at[sidx], sb, ls)
        pltpu.async_copy(ad_hbm.at[didx], db, ls)

    def _scale(exb, rows, r0, nrows):
        for g in range(nrows // _L):
            ex16 = exb[pl.ds(r0 + g * _L, _L)]
            for jj in range(_L):
                r = r0 + g * _L + jj
                exj = jnp.full((_L,), ex16[jj], jnp.float32)
                for kk in range(c // _L):
                    rows[r, pl.ds(kk * _L, _L)] = rows[r, pl.ds(kk * _L, _L)] * exj

    def _compute(k, cur):
        (sb, db, exb, dstb, dha, dhb, rows,
         gsa, gsb, ls, ds, fsa, fsb) = bufs[cur]
        sidxa = idx2.at[pl.ds(k * 2 * _B, _HA)]
        sidxb = idx2.at[pl.ds(k * 2 * _B + _HA, _HB)]
        sidx = idx2.at[pl.ds(k * 2 * _B, _B)]
        didx = idx2.at[pl.ds(k * 2 * _B + _B, _B)]
        pltpu.make_async_copy(as_hbm.at[sidx], sb, ls).wait()
        pltpu.make_async_copy(ad_hbm.at[didx], db, ls).wait()
        for j in range(_B // _L):
            z = sb[pl.ds(j * _L, _L)] + db[pl.ds(j * _L, _L)]
            e16 = jnp.where(z >= 0.0, z, jnp.float32(0.2) * z)
            exb[pl.ds(j * _L, _L)] = jnp.exp(e16 - m)
            dst16 = idx2[pl.ds(k * 2 * _B + _B + j * _L, _L)]
            dstb[pl.ds(j * _L, _L)] = dst16
            if j < _HA // _L:
                dha[pl.ds(j * _L, _L)] = dst16
            else:
                dhb[pl.ds((j - _HA // _L) * _L, _L)] = dst16
        pltpu.async_copy(exb, den_sh.at[dstb], ds, add=True)
        pltpu.make_async_copy(xp_hbm.at[sidxa], rows.at[pl.ds(0, _HA)],
                              gsa).wait()
        _scale(exb, rows, 0, _HA)
        pltpu.async_copy(rows.at[pl.ds(0, _HA)], num_sh.at[dha], fsa, add=True)
        pltpu.make_async_copy(xp_hbm.at[sidxb], rows.at[pl.ds(_HA, _HB)],
                              gsb).wait()
        _scale(exb, rows, _HA, _HB)
        pltpu.async_copy(rows.at[pl.ds(_HA, _HB)], num_sh.at[dhb], fsb,
                         add=True)

    _issue(0, 0, False)
    _issue(1, 1, False)
    _compute(0, 0)

    def _pair(i, carry):
        k0 = 1 + 2 * i
        _issue(k0 + 1, 0, True)
        _compute(k0, 1)
        _issue(k0 + 2, 1, True)
        _compute(k0 + 1, 0)
        return carry
    lax.fori_loop(0, (n_chunks - 3) // 2, _pair, 0)
    _issue(n_chunks - 1, 0, True)
    _compute(n_chunks - 2, 1)
    _compute(n_chunks - 1, 0)
    for p in range(2):
        (sb, db, exb, dstb, dha, dhb, rows,
         gsa, gsb, ls, ds, fsa, fsb) = bufs[p]
        pltpu.make_async_copy(rows.at[pl.ds(0, _HA)], num_sh.at[dha],
                              fsa).wait()
        pltpu.make_async_copy(rows.at[pl.ds(_HA, _HB)], num_sh.at[dhb],
                              fsb).wait()
        pltpu.make_async_copy(exb, den_sh.at[dstb], ds).wait()
    plsc.subcore_barrier()

    pltpu.sync_copy(num_sh.at[pl.ds(row0, rpt)],
                    num_out.at[ci, pl.ds(row0, rpt)])
    if rem_rows:
        @pl.when(si == _NS - 1)
        def _wb_tail():
            pltpu.sync_copy(num_sh.at[pl.ds(n - rem_rows, rem_rows)],
                            num_out.at[ci, pl.ds(n - rem_rows, rem_rows)])
    dbase = ci * n
    for t in range(full):
        pltpu.sync_copy(den_sh.at[pl.ds(row0 + t * _B, _B)], exb0)
        pltpu.sync_copy(exb0, den_out.at[pl.ds(dbase + row0 + t * _B, _B)])
    if rem:
        pltpu.sync_copy(den_sh.at[pl.ds(row0 + full * _B, rem)],
                        exb0.at[pl.ds(0, rem)])
        pltpu.sync_copy(exb0.at[pl.ds(0, rem)],
                        den_out.at[pl.ds(dbase + row0 + full * _B, rem)])
    if rem_rows:
        @pl.when(si == _NS - 1)
        def _wb_dtail():
            pltpu.sync_copy(den_sh.at[pl.ds(n - rem_rows, rem_rows)],
                            exb0.at[pl.ds(0, rem_rows)])
            pltpu.sync_copy(exb0.at[pl.ds(0, rem_rows)],
                            den_out.at[pl.ds(dbase + n - rem_rows, rem_rows)])


def _edge(xp, a_s, a_d, m16, packed, n_chunks):
    n, c = xp.shape
    e = _NW * n_chunks * _B
    mesh = plsc.VectorSubcoreMesh(core_axis_name="c", subcore_axis_name="s")
    kern = pl.kernel(
        functools.partial(_edge_body, n, e, c),
        out_type=(
            jax.ShapeDtypeStruct((_NC, n, c), jnp.float32),
            jax.ShapeDtypeStruct((_NC * n,), jnp.float32),
        ),
        mesh=mesh,
        compiler_params=pltpu.CompilerParams(needs_layout_passes=False),
        scratch_types=[
            pltpu.VMEM((n_chunks * 2 * _B,), jnp.int32),
            pltpu.VMEM((_B, c), jnp.float32),
            pltpu.VMEM((_B, c), jnp.float32),
            pltpu.VMEM((_B,), jnp.float32),
            pltpu.VMEM((_B,), jnp.float32),
            pltpu.VMEM((_B,), jnp.float32),
            pltpu.VMEM((_B,), jnp.int32),
            pltpu.VMEM((_HA,), jnp.int32),
            pltpu.VMEM((_HB,), jnp.int32),
            pltpu.VMEM((_B,), jnp.float32),
            pltpu.VMEM((_B,), jnp.float32),
            pltpu.VMEM((_B,), jnp.float32),
            pltpu.VMEM((_B,), jnp.int32),
            pltpu.VMEM((_HA,), jnp.int32),
            pltpu.VMEM((_HB,), jnp.int32),
            pltpu.VMEM((16,), jnp.float32),
            pltpu.VMEM_SHARED((n, c), jnp.float32),
            pltpu.VMEM_SHARED((n,), jnp.float32),
            pltpu.SemaphoreType.DMA,
            pltpu.SemaphoreType.DMA,
            pltpu.SemaphoreType.DMA,
            pltpu.SemaphoreType.DMA,
            pltpu.SemaphoreType.DMA,
            pltpu.SemaphoreType.DMA,
            pltpu.SemaphoreType.DMA,
            pltpu.SemaphoreType.DMA,
            pltpu.SemaphoreType.DMA,
            pltpu.SemaphoreType.DMA,
            pltpu.SemaphoreType.DMA,
            pltpu.SemaphoreType.DMA,
        ],
    )
    return kern(xp, a_s, a_d, m16, packed)


def _fin_body(num_ref, den_ref, bias_ref, out_ref):
    num = num_ref[0] + num_ref[1]
    den = jnp.sum(den_ref[0], axis=0) + jnp.float32(1e-16)
    o = num / den[:, None] + bias_ref[...][None, :]
    mx = jnp.max(o, axis=1, keepdims=True)
    lse = jnp.log(jnp.sum(jnp.exp(o - mx), axis=1, keepdims=True)) + mx
    out_ref[...] = o - lse


def _finalize(num_part, den_part, bias):
    _, n, c = num_part.shape
    blk = 1000
    grid = (n // blk,)
    den3 = den_part.reshape(_NC, n // blk, blk).transpose(1, 0, 2)
    return pl.pallas_call(
        _fin_body,
        grid=grid,
        in_specs=[
            pl.BlockSpec((_NC, blk, c), lambda i: (0, i, 0)),
            pl.BlockSpec((1, _NC, blk), lambda i: (i, 0, 0)),
            pl.BlockSpec((c,), lambda i: (0,)),
        ],
        out_specs=pl.BlockSpec((blk, c), lambda i: (i, 0)),
        out_shape=jax.ShapeDtypeStruct((n, c), jnp.float32),
    )(num_part, den3, bias)


def kernel(x, W, att_src, att_dst, bias, edge_index):
    e = edge_index.shape[1]
    n_chunks = e // (_NW * _B)
    packed = (edge_index.reshape(2, _NW, n_chunks, _B)
              .transpose(1, 2, 0, 3).reshape(_NW * n_chunks * 2 * _B))
    xp, a_s, a_d, m16 = _prep(x, W, att_src, att_dst)
    num_part, den_part = _edge(xp, a_s, a_d, m16, packed, n_chunks)
    return _finalize(num_part, den_part, bias)


# speedup vs baseline: 1.0550x; 1.0045x over previous
import functools

import jax
import jax.numpy as jnp
from jax import lax
from jax.experimental import pallas as pl
from jax.experimental.pallas import tpu as pltpu
from jax.experimental.pallas import tpu_sc as plsc

_NC = 2
_NS = 16
_NW = _NC * _NS
_L = 16
_B = 80


def _prep_body(x_ref, w_ref, asrc_ref, adst_ref, xp_ref, as_ref, ad_ref, m_ref):
    xp = jnp.dot(x_ref[...], w_ref[...], preferred_element_type=jnp.float32)
    xp_ref[0] = xp
    xp_ref[1] = xp
    a_s = jnp.sum(xp * asrc_ref[...][None, :], axis=1)
    a_d = jnp.sum(xp * adst_ref[...][None, :], axis=1)
    as_ref[...] = a_s
    ad_ref[...] = a_d
    z = jnp.max(a_s) + jnp.max(a_d)
    m = jnp.where(z >= 0.0, z, 0.2 * z)
    m_ref[...] = jnp.full((16,), m, jnp.float32)


def _prep(x, W, att_src, att_dst):
    n, d = x.shape
    c = W.shape[1]
    return pl.pallas_call(
        _prep_body,
        out_shape=(
            jax.ShapeDtypeStruct((2, n, c), jnp.float32),
            jax.ShapeDtypeStruct((n,), jnp.float32),
            jax.ShapeDtypeStruct((n,), jnp.float32),
            jax.ShapeDtypeStruct((16,), jnp.float32),
        ),
    )(x, W, att_src, att_dst)


_HA = 48
_HB = 32


def _edge_body(n, e, c, xp_hbm, as_hbm, ad_hbm, m_hbm, packed_hbm,
               num_out, den_out,
               idx2, rows0, rows1,
               sb0, db0, exb0, dstb0, dha0, dhb0,
               sb1, db1, exb1, dstb1, dha1, dhb1, m_v,
               num_sh, den_sh,
               gsa0, gsb0, ls0, ds0, fsa0, fsb0,
               gsa1, gsb1, ls1, ds1, fsa1, fsb1):
    ci = lax.axis_index("c")
    si = lax.axis_index("s")
    wid = si * _NC + ci
    e_per_tile = e // _NW
    n_chunks = e_per_tile // _B
    rpt = (n // _NS) // 8 * 8
    rem_rows = n - rpt * _NS

    pltpu.sync_copy(m_hbm, m_v)
    m = m_v[...]

    zeros16 = jnp.zeros((_L,), jnp.float32)
    for i in range(_B // _L):
        exb0[pl.ds(i * _L, _L)] = zeros16

    def _zero_rows(i, carry):
        for k in range(c // _L):
            rows0[i, pl.ds(k * _L, _L)] = zeros16
        return carry
    lax.fori_loop(0, _B, _zero_rows, 0)

    row0 = si * rpt
    full, rem = rpt // _B, rpt % _B
    for t in range(full):
        pltpu.sync_copy(rows0, num_sh.at[pl.ds(row0 + t * _B, _B)])
        pltpu.sync_copy(exb0, den_sh.at[pl.ds(row0 + t * _B, _B)])
    if rem:
        pltpu.sync_copy(rows0.at[pl.ds(0, rem)],
                        num_sh.at[pl.ds(row0 + full * _B, rem)])
        pltpu.sync_copy(exb0.at[pl.ds(0, rem)],
                        den_sh.at[pl.ds(row0 + full * _B, rem)])
    if rem_rows:
        @pl.when(si == _NS - 1)
        def _zero_tail():
            pltpu.sync_copy(rows0.at[pl.ds(0, rem_rows)],
                            num_sh.at[pl.ds(n - rem_rows, rem_rows)])
            pltpu.sync_copy(exb0.at[pl.ds(0, rem_rows)],
                            den_sh.at[pl.ds(n - rem_rows, rem_rows)])
    pltpu.sync_copy(packed_hbm.at[pl.ds(wid * n_chunks * 2 * _B,
                                        n_chunks * 2 * _B)], idx2)
    plsc.subcore_barrier()

    bufs = ((sb0, db0, exb0, dstb0, dha0, dhb0, rows0,
             gsa0, gsb0, ls0, ds0, fsa0, fsb0),
            (sb1, db1, exb1, dstb1, dha1, dhb1, rows1,
             gsa1, gsb1, ls1, ds1, fsa1, fsb1))

    def _issue(kn, nxt, wait_prev):
        (sb, db, exb, dstb, dha, dhb, rows,
         gsa, gsb, ls, ds, fsa, fsb) = bufs[nxt]
        if wait_prev:
            pltpu.make_async_copy(rows.at[pl.ds(0, _HA)],
                                  num_sh.at[dha], fsa).wait()
            pltpu.make_async_copy(rows.at[pl.ds(_HA, _HB)],
                                  num_sh.at[dhb], fsb).wait()
            pltpu.make_async_copy(exb, den_sh.at[dstb], ds).wait()
        sidxa = idx2.at[pl.ds(kn * 2 * _B, _HA)]
        sidxb = idx2.at[pl.ds(kn * 2 * _B + _HA, _HB)]
        sidx = idx2.at[pl.ds(kn * 2 * _B, _B)]
        didx = idx2.at[pl.ds(kn * 2 * _B + _B, _B)]
        pltpu.async_copy(xp_hbm.at[ci].at[sidxa], rows.at[pl.ds(0, _HA)], gsa)
        pltpu.async_copy(xp_hbm.at[ci].at[sidxb], rows.at[pl.ds(_HA, _HB)], gsb)
        pltpu.async_copy(as_hbm.at[sidx], sb, ls)
        pltpu.async_copy(ad_hbm.at[didx], db, ls)

    def _scale(exb, rows, r0, nrows):
        for g in range(nrows // _L):
            ex16 = exb[pl.ds(r0 + g * _L, _L)]
            for jj in range(_L):
                r = r0 + g * _L + jj
                exj = jnp.full((_L,), ex16[jj], jnp.float32)
                for kk in range(c // _L):
                    rows[r, pl.ds(kk * _L, _L)] = rows[r, pl.ds(kk * _L, _L)] * exj

    def _compute(k, cur):
        (sb, db, exb, dstb, dha, dhb, rows,
         gsa, gsb, ls, ds, fsa, fsb) = bufs[cur]
        sidxa = idx2.at[pl.ds(k * 2 * _B, _HA)]
        sidxb = idx2.at[pl.ds(k * 2 * _B + _HA, _HB)]
        sidx = idx2.at[pl.ds(k * 2 * _B, _B)]
        didx = idx2.at[pl.ds(k * 2 * _B + _B, _B)]
        pltpu.make_async_copy(as_hbm.at[sidx], sb, ls).wait()
        pltpu.make_async_copy(ad_hbm.at[didx], db, ls).wait()
        for j in range(_B // _L):
            z = sb[pl.ds(j * _L, _L)] + db[pl.ds(j * _L, _L)]
            e16 = jnp.where(z >= 0.0, z, jnp.float32(0.2) * z)
            exb[pl.ds(j * _L, _L)] = jnp.exp(e16 - m)
            dst16 = idx2[pl.ds(k * 2 * _B + _B + j * _L, _L)]
            dstb[pl.ds(j * _L, _L)] = dst16
            if j < _HA // _L:
                dha[pl.ds(j * _L, _L)] = dst16
            else:
                dhb[pl.ds((j - _HA // _L) * _L, _L)] = dst16
        pltpu.async_copy(exb, den_sh.at[dstb], ds, add=True)
        pltpu.make_async_copy(xp_hbm.at[ci].at[sidxa], rows.at[pl.ds(0, _HA)],
                              gsa).wait()
        _scale(exb, rows, 0, _HA)
        pltpu.async_copy(rows.at[pl.ds(0, _HA)], num_sh.at[dha], fsa, add=True)
        pltpu.make_async_copy(xp_hbm.at[ci].at[sidxb], rows.at[pl.ds(_HA, _HB)],
                              gsb).wait()
        _scale(exb, rows, _HA, _HB)
        pltpu.async_copy(rows.at[pl.ds(_HA, _HB)], num_sh.at[dhb], fsb,
                         add=True)

    _issue(0, 0, False)
    _issue(1, 1, False)
    _compute(0, 0)

    def _pair(i, carry):
        k0 = 1 + 2 * i
        _issue(k0 + 1, 0, True)
        _compute(k0, 1)
        _issue(k0 + 2, 1, True)
        _compute(k0 + 1, 0)
        return carry
    lax.fori_loop(0, (n_chunks - 3) // 2, _pair, 0)
    _issue(n_chunks - 1, 0, True)
    _compute(n_chunks - 2, 1)
    _compute(n_chunks - 1, 0)
    for p in range(2):
        (sb, db, exb, dstb, dha, dhb, rows,
         gsa, gsb, ls, ds, fsa, fsb) = bufs[p]
        pltpu.make_async_copy(rows.at[pl.ds(0, _HA)], num_sh.at[dha],
                              fsa).wait()
        pltpu.make_async_copy(rows.at[pl.ds(_HA, _HB)], num_sh.at[dhb],
                              fsb).wait()
        pltpu.make_async_copy(exb, den_sh.at[dstb], ds).wait()
    plsc.subcore_barrier()

    pltpu.sync_copy(num_sh.at[pl.ds(row0, rpt)],
                    num_out.at[ci, pl.ds(row0, rpt)])
    if rem_rows:
        @pl.when(si == _NS - 1)
        def _wb_tail():
            pltpu.sync_copy(num_sh.at[pl.ds(n - rem_rows, rem_rows)],
                            num_out.at[ci, pl.ds(n - rem_rows, rem_rows)])
    dbase = ci * n
    for t in range(full):
        pltpu.sync_copy(den_sh.at[pl.ds(row0 + t * _B, _B)], exb0)
        pltpu.sync_copy(exb0, den_out.at[pl.ds(dbase + row0 + t * _B, _B)])
    if rem:
        pltpu.sync_copy(den_sh.at[pl.ds(row0 + full * _B, rem)],
                        exb0.at[pl.ds(0, rem)])
        pltpu.sync_copy(exb0.at[pl.ds(0, rem)],
                        den_out.at[pl.ds(dbase + row0 + full * _B, rem)])
    if rem_rows:
        @pl.when(si == _NS - 1)
        def _wb_dtail():
            pltpu.sync_copy(den_sh.at[pl.ds(n - rem_rows, rem_rows)],
                            exb0.at[pl.ds(0, rem_rows)])
            pltpu.sync_copy(exb0.at[pl.ds(0, rem_rows)],
                            den_out.at[pl.ds(dbase + n - rem_rows, rem_rows)])


def _edge(xp, a_s, a_d, m16, packed, n_chunks):
    _, n, c = xp.shape
    e = _NW * n_chunks * _B
    mesh = plsc.VectorSubcoreMesh(core_axis_name="c", subcore_axis_name="s")
    kern = pl.kernel(
        functools.partial(_edge_body, n, e, c),
        out_type=(
            jax.ShapeDtypeStruct((_NC, n, c), jnp.float32),
            jax.ShapeDtypeStruct((_NC * n,), jnp.float32),
        ),
        mesh=mesh,
        compiler_params=pltpu.CompilerParams(needs_layout_passes=False),
        scratch_types=[
            pltpu.VMEM((n_chunks * 2 * _B,), jnp.int32),
            pltpu.VMEM((_B, c), jnp.float32),
            pltpu.VMEM((_B, c), jnp.float32),
            pltpu.VMEM((_B,), jnp.float32),
            pltpu.VMEM((_B,), jnp.float32),
            pltpu.VMEM((_B,), jnp.float32),
            pltpu.VMEM((_B,), jnp.int32),
            pltpu.VMEM((_HA,), jnp.int32),
            pltpu.VMEM((_HB,), jnp.int32),
            pltpu.VMEM((_B,), jnp.float32),
            pltpu.VMEM((_B,), jnp.float32),
            pltpu.VMEM((_B,), jnp.float32),
            pltpu.VMEM((_B,), jnp.int32),
            pltpu.VMEM((_HA,), jnp.int32),
            pltpu.VMEM((_HB,), jnp.int32),
            pltpu.VMEM((16,), jnp.float32),
            pltpu.VMEM_SHARED((n, c), jnp.float32),
            pltpu.VMEM_SHARED((n,), jnp.float32),
            pltpu.SemaphoreType.DMA,
            pltpu.SemaphoreType.DMA,
            pltpu.SemaphoreType.DMA,
            pltpu.SemaphoreType.DMA,
            pltpu.SemaphoreType.DMA,
            pltpu.SemaphoreType.DMA,
            pltpu.SemaphoreType.DMA,
            pltpu.SemaphoreType.DMA,
            pltpu.SemaphoreType.DMA,
            pltpu.SemaphoreType.DMA,
            pltpu.SemaphoreType.DMA,
            pltpu.SemaphoreType.DMA,
        ],
    )
    return kern(xp, a_s, a_d, m16, packed)


def _fin_body(num_ref, den_ref, bias_ref, out_ref):
    num = num_ref[0] + num_ref[1]
    den = jnp.sum(den_ref[0], axis=0) + jnp.float32(1e-16)
    o = num / den[:, None] + bias_ref[...][None, :]
    mx = jnp.max(o, axis=1, keepdims=True)
    lse = jnp.log(jnp.sum(jnp.exp(o - mx), axis=1, keepdims=True)) + mx
    out_ref[...] = o - lse


def _finalize(num_part, den_part, bias):
    _, n, c = num_part.shape
    blk = 1000
    grid = (n // blk,)
    den3 = den_part.reshape(_NC, n // blk, blk).transpose(1, 0, 2)
    return pl.pallas_call(
        _fin_body,
        grid=grid,
        in_specs=[
            pl.BlockSpec((_NC, blk, c), lambda i: (0, i, 0)),
            pl.BlockSpec((1, _NC, blk), lambda i: (i, 0, 0)),
            pl.BlockSpec((c,), lambda i: (0,)),
        ],
        out_specs=pl.BlockSpec((blk, c), lambda i: (i, 0)),
        out_shape=jax.ShapeDtypeStruct((n, c), jnp.float32),
    )(num_part, den3, bias)


def kernel(x, W, att_src, att_dst, bias, edge_index):
    e = edge_index.shape[1]
    n_chunks = e // (_NW * _B)
    packed = (edge_index.reshape(2, _NW, n_chunks, _B)
              .transpose(1, 2, 0, 3).reshape(_NW * n_chunks * 2 * _B))
    xp, a_s, a_d, m16 = _prep(x, W, att_src, att_dst)
    num_part, den_part = _edge(xp, a_s, a_d, m16, packed, n_chunks)
    return _finalize(num_part, den_part, bias)


# speedup vs baseline: 1.1360x; 1.0767x over previous
import functools

import jax
import jax.numpy as jnp
from jax import lax
from jax.experimental import pallas as pl
from jax.experimental.pallas import tpu as pltpu
from jax.experimental.pallas import tpu_sc as plsc

_NC = 2
_NS = 16
_NW = _NC * _NS
_L = 16
_B = 80


def _prep_body(x_ref, w_ref, asrc_ref, adst_ref, xp_ref, as_ref, ad_ref, m_ref):
    xp = jnp.dot(x_ref[...], w_ref[...], preferred_element_type=jnp.float32)
    xp_ref[...] = xp
    a_s = jnp.sum(xp * asrc_ref[...][None, :], axis=1)
    a_d = jnp.sum(xp * adst_ref[...][None, :], axis=1)
    as_ref[...] = a_s
    ad_ref[...] = a_d
    z = jnp.max(a_s) + jnp.max(a_d)
    m = jnp.where(z >= 0.0, z, 0.2 * z)
    m_ref[...] = jnp.full((16,), m, jnp.float32)


def _prep(x, W, att_src, att_dst):
    n, d = x.shape
    c = W.shape[1]
    return pl.pallas_call(
        _prep_body,
        out_shape=(
            jax.ShapeDtypeStruct((n, c), jnp.float32),
            jax.ShapeDtypeStruct((n,), jnp.float32),
            jax.ShapeDtypeStruct((n,), jnp.float32),
            jax.ShapeDtypeStruct((16,), jnp.float32),
        ),
    )(x, W, att_src, att_dst)


_HA = 48
_HB = 32


def _edge_body(n, e, c, xp_hbm, as_hbm, ad_hbm, m_hbm, ei_hbm,
               num_out, den_out,
               idxs, idxd, rows0, rows1,
               sb0, db0, exb0, dstb0, dha0, dhb0,
               sb1, db1, exb1, dstb1, dha1, dhb1, m_v,
               num_sh, den_sh,
               gsa0, gsb0, ls0, ds0, fsa0, fsb0,
               gsa1, gsb1, ls1, ds1, fsa1, fsb1):
    ci = lax.axis_index("c")
    si = lax.axis_index("s")
    wid = si * _NC + ci
    e_per_tile = e // _NW
    n_chunks = e_per_tile // _B
    rpt = (n // _NS) // 8 * 8
    rem_rows = n - rpt * _NS

    pltpu.sync_copy(m_hbm, m_v)
    m = m_v[...]

    zeros16 = jnp.zeros((_L,), jnp.float32)
    for i in range(_B // _L):
        exb0[pl.ds(i * _L, _L)] = zeros16

    def _zero_rows(i, carry):
        for k in range(c // _L):
            rows0[i, pl.ds(k * _L, _L)] = zeros16
        return carry
    lax.fori_loop(0, _B, _zero_rows, 0)

    row0 = si * rpt
    full, rem = rpt // _B, rpt % _B
    for t in range(full):
        pltpu.sync_copy(rows0, num_sh.at[pl.ds(row0 + t * _B, _B)])
        pltpu.sync_copy(exb0, den_sh.at[pl.ds(row0 + t * _B, _B)])
    if rem:
        pltpu.sync_copy(rows0.at[pl.ds(0, rem)],
                        num_sh.at[pl.ds(row0 + full * _B, rem)])
        pltpu.sync_copy(exb0.at[pl.ds(0, rem)],
                        den_sh.at[pl.ds(row0 + full * _B, rem)])
    if rem_rows:
        @pl.when(si == _NS - 1)
        def _zero_tail():
            pltpu.sync_copy(rows0.at[pl.ds(0, rem_rows)],
                            num_sh.at[pl.ds(n - rem_rows, rem_rows)])
            pltpu.sync_copy(exb0.at[pl.ds(0, rem_rows)],
                            den_sh.at[pl.ds(n - rem_rows, rem_rows)])
    pltpu.sync_copy(ei_hbm.at[pl.ds(wid * e_per_tile, e_per_tile)], idxs)
    pltpu.sync_copy(ei_hbm.at[pl.ds(e + wid * e_per_tile, e_per_tile)], idxd)
    plsc.subcore_barrier()

    bufs = ((sb0, db0, exb0, dstb0, dha0, dhb0, rows0,
             gsa0, gsb0, ls0, ds0, fsa0, fsb0),
            (sb1, db1, exb1, dstb1, dha1, dhb1, rows1,
             gsa1, gsb1, ls1, ds1, fsa1, fsb1))

    def _issue(kn, nxt, wait_prev):
        (sb, db, exb, dstb, dha, dhb, rows,
         gsa, gsb, ls, ds, fsa, fsb) = bufs[nxt]
        if wait_prev:
            pltpu.make_async_copy(rows.at[pl.ds(0, _HA)],
                                  num_sh.at[dha], fsa).wait()
            pltpu.make_async_copy(rows.at[pl.ds(_HA, _HB)],
                                  num_sh.at[dhb], fsb).wait()
            pltpu.make_async_copy(exb, den_sh.at[dstb], ds).wait()
        sidxa = idxs.at[pl.ds(kn * _B, _HA)]
        sidxb = idxs.at[pl.ds(kn * _B + _HA, _HB)]
        sidx = idxs.at[pl.ds(kn * _B, _B)]
        didx = idxd.at[pl.ds(kn * _B, _B)]
        pltpu.async_copy(xp_hbm.at[sidxa], rows.at[pl.ds(0, _HA)], gsa)
        pltpu.async_copy(xp_hbm.at[sidxb], rows.at[pl.ds(_HA, _HB)], gsb)
        pltpu.async_copy(as_hbm.at[sidx], sb, ls)
        pltpu.async_copy(ad_hbm.at[didx], db, ls)

    def _scale(exb, rows, r0, nrows):
        for g in range(nrows // _L):
            ex16 = exb[pl.ds(r0 + g * _L, _L)]
            for jj in range(_L):
                r = r0 + g * _L + jj
                exj = jnp.full((_L,), ex16[jj], jnp.float32)
                for kk in range(c // _L):
                    rows[r, pl.ds(kk * _L, _L)] = rows[r, pl.ds(kk * _L, _L)] * exj

    def _compute(k, cur):
        (sb, db, exb, dstb, dha, dhb, rows,
         gsa, gsb, ls, ds, fsa, fsb) = bufs[cur]
        sidxa = idxs.at[pl.ds(k * _B, _HA)]
        sidxb = idxs.at[pl.ds(k * _B + _HA, _HB)]
        sidx = idxs.at[pl.ds(k * _B, _B)]
        didx = idxd.at[pl.ds(k * _B, _B)]
        pltpu.make_async_copy(as_hbm.at[sidx], sb, ls).wait()
        pltpu.make_async_copy(ad_hbm.at[didx], db, ls).wait()
        for j in range(_B // _L):
            z = sb[pl.ds(j * _L, _L)] + db[pl.ds(j * _L, _L)]
            e16 = jnp.where(z >= 0.0, z, jnp.float32(0.2) * z)
            exb[pl.ds(j * _L, _L)] = jnp.exp(e16 - m)
            dst16 = idxd[pl.ds(k * _B + j * _L, _L)]
            dstb[pl.ds(j * _L, _L)] = dst16
            if j < _HA // _L:
                dha[pl.ds(j * _L, _L)] = dst16
            else:
                dhb[pl.ds((j - _HA // _L) * _L, _L)] = dst16
        pltpu.async_copy(exb, den_sh.at[dstb], ds, add=True)
        pltpu.make_async_copy(xp_hbm.at[sidxa], rows.at[pl.ds(0, _HA)],
                              gsa).wait()
        _scale(exb, rows, 0, _HA)
        pltpu.async_copy(rows.at[pl.ds(0, _HA)], num_sh.at[dha], fsa, add=True)
        pltpu.make_async_copy(xp_hbm.at[sidxb], rows.at[pl.ds(_HA, _HB)],
                              gsb).wait()
        _scale(exb, rows, _HA, _HB)
        pltpu.async_copy(rows.at[pl.ds(_HA, _HB)], num_sh.at[dhb], fsb,
                         add=True)

    _issue(0, 0, False)
    _issue(1, 1, False)
    _compute(0, 0)

    def _pair(i, carry):
        k0 = 1 + 2 * i
        _issue(k0 + 1, 0, True)
        _compute(k0, 1)
        _issue(k0 + 2, 1, True)
        _compute(k0 + 1, 0)
        return carry
    lax.fori_loop(0, (n_chunks - 3) // 2, _pair, 0)
    _issue(n_chunks - 1, 0, True)
    _compute(n_chunks - 2, 1)
    _compute(n_chunks - 1, 0)
    for p in range(2):
        (sb, db, exb, dstb, dha, dhb, rows,
         gsa, gsb, ls, ds, fsa, fsb) = bufs[p]
        pltpu.make_async_copy(rows.at[pl.ds(0, _HA)], num_sh.at[dha],
                              fsa).wait()
        pltpu.make_async_copy(rows.at[pl.ds(_HA, _HB)], num_sh.at[dhb],
                              fsb).wait()
        pltpu.make_async_copy(exb, den_sh.at[dstb], ds).wait()
    plsc.subcore_barrier()

    pltpu.sync_copy(num_sh.at[pl.ds(row0, rpt)],
                    num_out.at[ci, pl.ds(row0, rpt)])
    if rem_rows:
        @pl.when(si == _NS - 1)
        def _wb_tail():
            pltpu.sync_copy(num_sh.at[pl.ds(n - rem_rows, rem_rows)],
                            num_out.at[ci, pl.ds(n - rem_rows, rem_rows)])
    dbase = ci * n
    for t in range(full):
        pltpu.sync_copy(den_sh.at[pl.ds(row0 + t * _B, _B)], exb0)
        pltpu.sync_copy(exb0, den_out.at[pl.ds(dbase + row0 + t * _B, _B)])
    if rem:
        pltpu.sync_copy(den_sh.at[pl.ds(row0 + full * _B, rem)],
                        exb0.at[pl.ds(0, rem)])
        pltpu.sync_copy(exb0.at[pl.ds(0, rem)],
                        den_out.at[pl.ds(dbase + row0 + full * _B, rem)])
    if rem_rows:
        @pl.when(si == _NS - 1)
        def _wb_dtail():
            pltpu.sync_copy(den_sh.at[pl.ds(n - rem_rows, rem_rows)],
                            exb0.at[pl.ds(0, rem_rows)])
            pltpu.sync_copy(exb0.at[pl.ds(0, rem_rows)],
                            den_out.at[pl.ds(dbase + n - rem_rows, rem_rows)])


def _edge(xp, a_s, a_d, m16, packed, n_chunks):
    n, c = xp.shape
    e = _NW * n_chunks * _B
    mesh = plsc.VectorSubcoreMesh(core_axis_name="c", subcore_axis_name="s")
    kern = pl.kernel(
        functools.partial(_edge_body, n, e, c),
        out_type=(
            jax.ShapeDtypeStruct((_NC, n, c), jnp.float32),
            jax.ShapeDtypeStruct((_NC * n,), jnp.float32),
        ),
        mesh=mesh,
        compiler_params=pltpu.CompilerParams(needs_layout_passes=False),
        scratch_types=[
            pltpu.VMEM((n_chunks * _B,), jnp.int32),
            pltpu.VMEM((n_chunks * _B,), jnp.int32),
            pltpu.VMEM((_B, c), jnp.float32),
            pltpu.VMEM((_B, c), jnp.float32),
            pltpu.VMEM((_B,), jnp.float32),
            pltpu.VMEM((_B,), jnp.float32),
            pltpu.VMEM((_B,), jnp.float32),
            pltpu.VMEM((_B,), jnp.int32),
            pltpu.VMEM((_HA,), jnp.int32),
            pltpu.VMEM((_HB,), jnp.int32),
            pltpu.VMEM((_B,), jnp.float32),
            pltpu.VMEM((_B,), jnp.float32),
            pltpu.VMEM((_B,), jnp.float32),
            pltpu.VMEM((_B,), jnp.int32),
            pltpu.VMEM((_HA,), jnp.int32),
            pltpu.VMEM((_HB,), jnp.int32),
            pltpu.VMEM((16,), jnp.float32),
            pltpu.VMEM_SHARED((n, c), jnp.float32),
            pltpu.VMEM_SHARED((n,), jnp.float32),
            pltpu.SemaphoreType.DMA,
            pltpu.SemaphoreType.DMA,
            pltpu.SemaphoreType.DMA,
            pltpu.SemaphoreType.DMA,
            pltpu.SemaphoreType.DMA,
            pltpu.SemaphoreType.DMA,
            pltpu.SemaphoreType.DMA,
            pltpu.SemaphoreType.DMA,
            pltpu.SemaphoreType.DMA,
            pltpu.SemaphoreType.DMA,
            pltpu.SemaphoreType.DMA,
            pltpu.SemaphoreType.DMA,
        ],
    )
    return kern(xp, a_s, a_d, m16, packed)


def _fin_body(num_ref, den_ref, bias_ref, out_ref):
    num = num_ref[0] + num_ref[1]
    den = jnp.sum(den_ref[0], axis=0) + jnp.float32(1e-16)
    o = num / den[:, None] + bias_ref[...][None, :]
    mx = jnp.max(o, axis=1, keepdims=True)
    lse = jnp.log(jnp.sum(jnp.exp(o - mx), axis=1, keepdims=True)) + mx
    out_ref[...] = o - lse


def _finalize(num_part, den_part, bias):
    _, n, c = num_part.shape
    blk = 1000
    grid = (n // blk,)
    den3 = den_part.reshape(_NC, n // blk, blk).transpose(1, 0, 2)
    return pl.pallas_call(
        _fin_body,
        grid=grid,
        in_specs=[
            pl.BlockSpec((_NC, blk, c), lambda i: (0, i, 0)),
            pl.BlockSpec((1, _NC, blk), lambda i: (i, 0, 0)),
            pl.BlockSpec((c,), lambda i: (0,)),
        ],
        out_specs=pl.BlockSpec((blk, c), lambda i: (i, 0)),
        out_shape=jax.ShapeDtypeStruct((n, c), jnp.float32),
    )(num_part, den3, bias)


def kernel(x, W, att_src, att_dst, bias, edge_index):
    e = edge_index.shape[1]
    n_chunks = e // (_NW * _B)
    packed = edge_index.reshape(2 * e)
    xp, a_s, a_d, m16 = _prep(x, W, att_src, att_dst)
    num_part, den_part = _edge(xp, a_s, a_d, m16, packed, n_chunks)
    return _finalize(num_part, den_part, bias)
